# Initial kernel scaffold; baseline (speedup 1.0000x reference)
import functools
import jax
import jax.numpy as jnp
from jax import lax
from jax.experimental import pallas as pl
from jax.experimental.pallas import tpu as pltpu, tpu_sc as plsc

NC = 2
NS = 16
NW = NC * NS
D = 128
CHUNK = 128
NBUF = 4


def _gather_kernel(idx_hbm, table_hbm, out_hbm, idx_v, bufs, gsem, wsem):
    wid = lax.axis_index("s") * NC + lax.axis_index("c")
    j_steps = idx_v.shape[0]
    base = wid * (j_steps * CHUNK)

    pltpu.sync_copy(idx_hbm.at[wid], idx_v)

    gathers = [None] * j_steps
    writes = [None] * j_steps

    def start_gather(j):
        return pltpu.async_copy(
            table_hbm.at[idx_v.at[j]], bufs.at[j % NBUF], gsem)

    for j in range(min(NBUF - 1, j_steps)):
        gathers[j] = start_gather(j)

    for j in range(j_steps):
        nj = j + NBUF - 1
        if nj < j_steps:
            if j >= 1:
                writes[j - 1].wait()
            gathers[nj] = start_gather(nj)
        gathers[j].wait()
        writes[j] = pltpu.async_copy(
            bufs.at[j % NBUF], out_hbm.at[pl.ds(base + j * CHUNK, CHUNK)],
            wsem)

    first_undrained = max(0, min(j_steps - (NBUF - 1), j_steps))
    if j_steps < NBUF:
        first_undrained = 0
    for j in range(first_undrained, j_steps):
        writes[j].wait()


@jax.jit
def _embedding_lookup(idx3, table):
    n_rows = idx3.shape[0] * idx3.shape[1] * idx3.shape[2]
    j_steps = idx3.shape[1]
    mesh = plsc.VectorSubcoreMesh(core_axis_name="c", subcore_axis_name="s")
    return pl.kernel(
        _gather_kernel,
        out_type=jax.ShapeDtypeStruct((n_rows, D), jnp.float32),
        mesh=mesh,
        scratch_types=[
            pltpu.VMEM((j_steps, CHUNK), jnp.int32),
            pltpu.VMEM((NBUF, CHUNK, D), jnp.float32),
            pltpu.SemaphoreType.DMA,
            pltpu.SemaphoreType.DMA,
        ],
    )(idx3, table)


def kernel(batch_original_ids, embedding_weight):
    b, t = batch_original_ids.shape
    idx3 = batch_original_ids.reshape(NW, (b * t) // (NW * CHUNK), CHUNK)
    out = _embedding_lookup(idx3, embedding_weight)
    return out.reshape(b, t, embedding_weight.shape[1])


# speedup vs baseline: 1.2986x; 1.2986x over previous
import functools
import jax
import jax.numpy as jnp
from jax import lax
from jax.experimental import pallas as pl
from jax.experimental.pallas import tpu as pltpu, tpu_sc as plsc

NC = 2
NS = 16
NW = NC * NS
D = 128
CHUNK = 128
NBUF = 4


def _gather_kernel(idx_hbm, table_hbm, out_hbm, idx_v, bufs, gsem, wsem):
    wid = lax.axis_index("s") * NC + lax.axis_index("c")
    j_steps = idx_v.shape[0]
    base = wid * (j_steps * CHUNK)

    pltpu.sync_copy(idx_hbm.at[wid], idx_v)

    gathers = [None] * j_steps
    writes = [None] * j_steps

    def start_gather(j):
        return pltpu.async_copy(
            table_hbm.at[idx_v.at[j]], bufs.at[j % NBUF], gsem)

    for j in range(min(NBUF - 1, j_steps)):
        gathers[j] = start_gather(j)

    for j in range(j_steps):
        nj = j + NBUF - 1
        if nj < j_steps:
            if j >= 1:
                writes[j - 1].wait()
            gathers[nj] = start_gather(nj)
        gathers[j].wait()
        writes[j] = pltpu.async_copy(
            bufs.at[j % NBUF], out_hbm.at[pl.ds(base + j * CHUNK, CHUNK)],
            wsem)

    for j in range(max(0, j_steps - NBUF), j_steps):
        writes[j].wait()


@jax.jit
def _embedding_lookup(idx3, table):
    n_rows = idx3.shape[0] * idx3.shape[1] * idx3.shape[2]
    j_steps = idx3.shape[1]
    mesh = plsc.VectorSubcoreMesh(core_axis_name="c", subcore_axis_name="s")
    return pl.kernel(
        _gather_kernel,
        out_type=jax.ShapeDtypeStruct((n_rows, D), jnp.float32),
        mesh=mesh,
        scratch_types=[
            pltpu.VMEM((j_steps, CHUNK), jnp.int32),
            pltpu.VMEM((NBUF, CHUNK, D), jnp.float32),
            pltpu.SemaphoreType.DMA,
            pltpu.SemaphoreType.DMA,
        ],
    )(idx3, table)


def kernel(batch_original_ids, embedding_weight):
    b, t = batch_original_ids.shape
    idx3 = batch_original_ids.reshape(NW, (b * t) // (NW * CHUNK), CHUNK)
    out = _embedding_lookup(idx3, embedding_weight)
    return out.reshape(b, t, embedding_weight.shape[1])


# speedup vs baseline: 1.3009x; 1.0017x over previous
import functools
import jax
import jax.numpy as jnp
from jax import lax
from jax.experimental import pallas as pl
from jax.experimental.pallas import tpu as pltpu, tpu_sc as plsc

NC = 2
NS = 16
NW = NC * NS
D = 128
CHUNK = 128
NBUF = 6


def _gather_kernel(idx_hbm, table_hbm, out_hbm, idx_v, bufs, gsem, wsem):
    wid = lax.axis_index("s") * NC + lax.axis_index("c")
    j_steps = idx_v.shape[0]
    base = wid * (j_steps * CHUNK)

    pltpu.sync_copy(idx_hbm.at[wid], idx_v)

    gathers = [None] * j_steps
    writes = [None] * j_steps

    def start_gather(j):
        return pltpu.async_copy(
            table_hbm.at[idx_v.at[j]], bufs.at[j % NBUF], gsem)

    for j in range(min(NBUF - 1, j_steps)):
        gathers[j] = start_gather(j)

    for j in range(j_steps):
        nj = j + NBUF - 1
        if nj < j_steps:
            if j >= 1:
                writes[j - 1].wait()
            gathers[nj] = start_gather(nj)
        gathers[j].wait()
        writes[j] = pltpu.async_copy(
            bufs.at[j % NBUF], out_hbm.at[pl.ds(base + j * CHUNK, CHUNK)],
            wsem)

    for j in range(max(0, j_steps - NBUF), j_steps):
        writes[j].wait()


@jax.jit
def _embedding_lookup(idx3, table):
    n_rows = idx3.shape[0] * idx3.shape[1] * idx3.shape[2]
    j_steps = idx3.shape[1]
    mesh = plsc.VectorSubcoreMesh(core_axis_name="c", subcore_axis_name="s")
    return pl.kernel(
        _gather_kernel,
        out_type=jax.ShapeDtypeStruct((n_rows, D), jnp.float32),
        mesh=mesh,
        scratch_types=[
            pltpu.VMEM((j_steps, CHUNK), jnp.int32),
            pltpu.VMEM((NBUF, CHUNK, D), jnp.float32),
            pltpu.SemaphoreType.DMA,
            pltpu.SemaphoreType.DMA,
        ],
    )(idx3, table)


def kernel(batch_original_ids, embedding_weight):
    b, t = batch_original_ids.shape
    idx3 = batch_original_ids.reshape(NW, (b * t) // (NW * CHUNK), CHUNK)
    out = _embedding_lookup(idx3, embedding_weight)
    return out.reshape(b, t, embedding_weight.shape[1])
